# baseline (speedup 1.0000x reference)
import functools

import jax
import jax.numpy as jnp
from jax import lax
from jax.experimental import pallas as pl
from jax.experimental.pallas import tpu as pltpu
from jax.experimental.pallas import tpu_sc as plsc

N = 10000
E = 160000
D = 256
H = 128
NC = 2
NS = 16
EPT = E // NS
K = 80
NCHUNK = EPT // K
BATCH = 5
NBATCH = NCHUNK // BATCH
NSLOT = 3
RPT = 624
TAIL = N - NS * RPT

BM = 2000
NB = N // BM


def _tc_body(x_ref, w1_ref, w2_ref, b1_ref, b2_ref, a1w_ref, a2w_ref,
             a1b_ref, a2b_ref, xi_ref, xj_ref, a1_ref, a2_ref):
    h = pl.program_id(1)
    x = x_ref[...]
    xi = jnp.maximum(
        jnp.dot(x, w1_ref[...], preferred_element_type=jnp.float32)
        + b1_ref[...], 0.0)
    xj = jnp.maximum(
        jnp.dot(x, w2_ref[...], preferred_element_type=jnp.float32)
        + b2_ref[...], 0.0)
    xi_ref[...] = xi
    xj_ref[...] = xj
    pa1 = jnp.dot(xi, a1w_ref[...], preferred_element_type=jnp.float32)
    pa2 = jnp.dot(xj, a2w_ref[...], preferred_element_type=jnp.float32)

    @pl.when(h == 0)
    def _():
        a1_ref[...] = pa1 + a1b_ref[...]
        a2_ref[...] = pa2 + a2b_ref[...]

    @pl.when(h != 0)
    def _():
        a1_ref[...] += pa1
        a2_ref[...] += pa2


def _tc_dense(x0, w1t, w2t, b1r, b2r, a1wt, a2wt, a1br, a2br):
    return pl.pallas_call(
        _tc_body,
        grid=(NB, 2),
        in_specs=[
            pl.BlockSpec((BM, D), lambda i, h: (i, 0)),
            pl.BlockSpec((D, H), lambda i, h: (0, h)),
            pl.BlockSpec((D, H), lambda i, h: (0, h)),
            pl.BlockSpec((1, H), lambda i, h: (0, h)),
            pl.BlockSpec((1, H), lambda i, h: (0, h)),
            pl.BlockSpec((H, 1), lambda i, h: (h, 0)),
            pl.BlockSpec((H, 1), lambda i, h: (h, 0)),
            pl.BlockSpec((1, 1), lambda i, h: (0, 0)),
            pl.BlockSpec((1, 1), lambda i, h: (0, 0)),
        ],
        out_specs=[
            pl.BlockSpec((BM, H), lambda i, h: (h * NB + i, 0)),
            pl.BlockSpec((BM, H), lambda i, h: (h * NB + i, 0)),
            pl.BlockSpec((BM, 1), lambda i, h: (i, 0)),
            pl.BlockSpec((BM, 1), lambda i, h: (i, 0)),
        ],
        out_shape=[
            jax.ShapeDtypeStruct((2 * N, H), jnp.float32),
            jax.ShapeDtypeStruct((2 * N, H), jnp.float32),
            jax.ShapeDtypeStruct((N, 1), jnp.float32),
            jax.ShapeDtypeStruct((N, 1), jnp.float32),
        ],
    )(x0, w1t, w2t, b1r, b2r, a1wt, a2wt, a1br, a2br)


def _sc_body(xj_hbm, xi_hbm, ei_hbm, a1p_hbm, a2_hbm, out_hbm,
             row_v, col_v,
             colo0, colo1, colo2, az1, ab20, ab21, ab22,
             gbuf0, gbuf1, gbuf2, acc,
             sem_ir, sem_ic, sem_a20, sem_a21, sem_a22,
             sem_x0, sem_x1, sem_x2, sem_s0, sem_s1, sem_s2):
    c = lax.axis_index("c")
    s = lax.axis_index("s")
    coff = c * N

    colo = (colo0, colo1, colo2)
    ab2 = (ab20, ab21, ab22)
    gbuf = (gbuf0, gbuf1, gbuf2)
    sem_a2 = (sem_a20, sem_a21, sem_a22)
    sem_x = (sem_x0, sem_x1, sem_x2)
    sem_s = (sem_s0, sem_s1, sem_s2)

    pltpu.sync_copy(a1p_hbm, az1)

    pltpu.sync_copy(
        xi_hbm.at[pl.ds(c * N + s * RPT, RPT)],
        acc.at[pl.ds(s * RPT, RPT)])

    @pl.when(s == NS - 1)
    def _():
        pltpu.sync_copy(
            xi_hbm.at[pl.ds(c * N + NS * RPT, TAIL)],
            acc.at[pl.ds(NS * RPT, TAIL)])

    plsc.subcore_barrier()

    def prep(u, sl):
        bt = u // BATCH
        jt = lax.rem(u, BATCH)
        sb = lax.rem(bt, 2)
        rb = sb * BATCH + jt

        @pl.when(u >= NSLOT)
        def _():
            pltpu.make_async_copy(
                gbuf[sl], acc.at[row_v.at[rb]], sem_s[sl]).wait()

        @pl.when(jt == 0)
        def _():
            pltpu.make_async_copy(ei_hbm.at[0, s, bt],
                                  row_v.at[pl.ds(sb * BATCH, BATCH)],
                                  sem_ir).wait()
            pltpu.make_async_copy(ei_hbm.at[1, s, bt],
                                  col_v.at[pl.ds(sb * BATCH, BATCH)],
                                  sem_ic).wait()

        @pl.when(jnp.logical_and(jt == 2, bt + 1 < NBATCH))
        def _():
            nb = bt + 1
            nsb = lax.rem(nb, 2)
            pltpu.async_copy(ei_hbm.at[0, s, nb],
                             row_v.at[pl.ds(nsb * BATCH, BATCH)], sem_ir)
            pltpu.async_copy(ei_hbm.at[1, s, nb],
                             col_v.at[pl.ds(nsb * BATCH, BATCH)], sem_ic)

        pltpu.async_copy(a2_hbm.at[col_v.at[rb]], ab2[sl], sem_a2[sl])

        for g in range(K // 16):
            colo[sl][pl.ds(g * 16, 16)] = (
                col_v[rb, pl.ds(g * 16, 16)] + coff)

        pltpu.async_copy(xj_hbm.at[colo[sl]], gbuf[sl], sem_x[sl])

    def work(t, sl):
        bt = t // BATCH
        jt = lax.rem(t, BATCH)
        sb = lax.rem(bt, 2)
        rb = sb * BATCH + jt

        pltpu.make_async_copy(a2_hbm.at[col_v.at[rb]], ab2[sl],
                              sem_a2[sl]).wait()
        pltpu.make_async_copy(xj_hbm.at[colo[sl]], gbuf[sl],
                              sem_x[sl]).wait()

        def g_body(g, _):
            ridx = row_v[rb, pl.ds(g * 16, 16)]
            pv = plsc.load_gather(az1, [lax.shift_right_logical(ridx, 1)])
            sh = lax.shift_left(1 - (ridx & 1), 4)
            a1g = plsc.bitcast(
                lax.shift_left(pv, sh) & jnp.int32(-65536), jnp.float32)
            a2g = ab2[sl][pl.ds(g * 16, 16)]
            att = 1.0 / (1.0 + jnp.exp(-(a1g + a2g)))
            for l in range(16):
                av = jnp.broadcast_to(att[l], (16,))
                k = g * 16 + l
                for d in range(H // 16):
                    gbuf[sl][k, pl.ds(d * 16, 16)] = (
                        gbuf[sl][k, pl.ds(d * 16, 16)] * av)
            return 0

        lax.fori_loop(0, K // 16, g_body, 0)

        pltpu.async_copy(gbuf[sl], acc.at[row_v.at[rb]], sem_s[sl],
                         add=True)

    pltpu.async_copy(ei_hbm.at[0, s, 0], row_v.at[pl.ds(0, BATCH)], sem_ir)
    pltpu.async_copy(ei_hbm.at[1, s, 0], col_v.at[pl.ds(0, BATCH)], sem_ic)
    prep(jnp.int32(0), 0)
    prep(jnp.int32(1), 1)

    def loop_body(i, _):
        for b3 in range(NSLOT):
            t = NSLOT * i + b3
            pl.when(t < NCHUNK)(lambda: work(t, b3))
            pl.when(t + 2 < NCHUNK)(
                lambda: prep(t + 2, (b3 + 2) % NSLOT))
        return 0

    lax.fori_loop(0, (NCHUNK + NSLOT - 1) // NSLOT, loop_body, 0)

    for x in range(NCHUNK - NSLOT, NCHUNK):
        sl = x % NSLOT
        pltpu.make_async_copy(gbuf[sl], acc.at[row_v.at[0]],
                              sem_s[sl]).wait()

    plsc.subcore_barrier()

    pltpu.sync_copy(
        acc.at[pl.ds(s * RPT, RPT)],
        out_hbm.at[pl.ds(s * RPT, RPT), pl.ds(c * H, H)])

    @pl.when(s == NS - 1)
    def _():
        pltpu.sync_copy(
            acc.at[pl.ds(NS * RPT, TAIL)],
            out_hbm.at[pl.ds(NS * RPT, TAIL), pl.ds(c * H, H)])


@functools.cache
def _sc_segment():
    return pl.kernel(
        _sc_body,
        out_type=jax.ShapeDtypeStruct((N, D), jnp.float32),
        mesh=plsc.VectorSubcoreMesh(core_axis_name="c", subcore_axis_name="s"),
        scratch_types=(
            [
                pltpu.VMEM((2 * BATCH, K), jnp.int32),
                pltpu.VMEM((2 * BATCH, K), jnp.int32),
            ]
            + [pltpu.VMEM((K,), jnp.int32)] * 3
            + [pltpu.VMEM((N // 2,), jnp.int32)]
            + [pltpu.VMEM((K,), jnp.float32)] * 3
            + [pltpu.VMEM((K, H), jnp.float32)] * 3
            + [pltpu.VMEM_SHARED((N, H), jnp.float32)]
            + [pltpu.SemaphoreType.DMA] * 11
        ),
        compiler_params=pltpu.CompilerParams(needs_layout_passes=False),
    )


def kernel(x0, x1, edge_index, W1, b1, W2, b2, a1_w, a1_b, a2_w, a2_b):
    ei = edge_index.astype(jnp.int32).reshape(2, NS, NBATCH, BATCH, K)
    xi2, xj2, a1n, a2n = _tc_dense(
        x0, W1.T, W2.T,
        b1.reshape(1, D), b2.reshape(1, D),
        a1_w.T, a2_w.T,
        a1_b.reshape(1, 1), a2_b.reshape(1, 1))
    a1p = lax.bitcast_convert_type(
        a1n.reshape(N // 2, 2).astype(jnp.bfloat16), jnp.int32)
    return _sc_segment()(xj2, xi2, ei, a1p, a2n.reshape(N))


# speedup vs baseline: 1.0591x; 1.0591x over previous
import functools

import jax
import jax.numpy as jnp
from jax import lax
from jax.experimental import pallas as pl
from jax.experimental.pallas import tpu as pltpu
from jax.experimental.pallas import tpu_sc as plsc

N = 10000
E = 160000
D = 256
H = 128
NC = 2
NS = 16
EPT = E // NS
K = 80
NCHUNK = EPT // K
BATCH = 5
NBATCH = NCHUNK // BATCH
NSLOT = 3
RPT = 624
TAIL = N - NS * RPT

BM = 2000
NB = N // BM


def _tc_body(x_ref, w1_ref, w2_ref, b1_ref, b2_ref, a1w_ref, a2w_ref,
             a1b_ref, a2b_ref, xi_ref, xj_ref, a1_ref, a2_ref):
    h = pl.program_id(1)
    x = x_ref[...]
    xi = jnp.maximum(
        jnp.dot(x, w1_ref[...], preferred_element_type=jnp.float32)
        + b1_ref[...], 0.0)
    xj = jnp.maximum(
        jnp.dot(x, w2_ref[...], preferred_element_type=jnp.float32)
        + b2_ref[...], 0.0)
    xi_ref[...] = xi
    xj_ref[...] = xj
    pa1 = jnp.dot(xi, a1w_ref[...], preferred_element_type=jnp.float32)
    pa2 = jnp.dot(xj, a2w_ref[...], preferred_element_type=jnp.float32)

    @pl.when(h == 0)
    def _():
        a1_ref[...] = pa1 + a1b_ref[...]
        a2_ref[...] = pa2 + a2b_ref[...]

    @pl.when(h != 0)
    def _():
        a1_ref[...] += pa1
        a2_ref[...] += pa2


def _tc_dense(x0, w1t, w2t, b1r, b2r, a1wt, a2wt, a1br, a2br):
    return pl.pallas_call(
        _tc_body,
        grid=(NB, 2),
        in_specs=[
            pl.BlockSpec((BM, D), lambda i, h: (i, 0)),
            pl.BlockSpec((D, H), lambda i, h: (0, h)),
            pl.BlockSpec((D, H), lambda i, h: (0, h)),
            pl.BlockSpec((1, H), lambda i, h: (0, h)),
            pl.BlockSpec((1, H), lambda i, h: (0, h)),
            pl.BlockSpec((H, 1), lambda i, h: (h, 0)),
            pl.BlockSpec((H, 1), lambda i, h: (h, 0)),
            pl.BlockSpec((1, 1), lambda i, h: (0, 0)),
            pl.BlockSpec((1, 1), lambda i, h: (0, 0)),
        ],
        out_specs=[
            pl.BlockSpec((BM, H), lambda i, h: (h * NB + i, 0)),
            pl.BlockSpec((BM, H), lambda i, h: (h * NB + i, 0)),
            pl.BlockSpec((BM, 1), lambda i, h: (i, 0)),
            pl.BlockSpec((BM, 1), lambda i, h: (i, 0)),
        ],
        out_shape=[
            jax.ShapeDtypeStruct((2 * N, H), jnp.float32),
            jax.ShapeDtypeStruct((2 * N, H), jnp.float32),
            jax.ShapeDtypeStruct((N, 1), jnp.float32),
            jax.ShapeDtypeStruct((N, 1), jnp.float32),
        ],
    )(x0, w1t, w2t, b1r, b2r, a1wt, a2wt, a1br, a2br)


def _sc_body(xj_hbm, xi_hbm, ei_hbm, a1_hbm, a2_hbm, out_hbm,
             row_v, col_v,
             colo0, colo1, colo2, ab10, ab11, ab12, ab20, ab21, ab22,
             gbuf0, gbuf1, gbuf2, acc,
             sem_ir, sem_ic,
             sem_a10, sem_a11, sem_a12, sem_a20, sem_a21, sem_a22,
             sem_x0, sem_x1, sem_x2, sem_s0, sem_s1, sem_s2):
    c = lax.axis_index("c")
    s = lax.axis_index("s")
    coff = c * N

    colo = (colo0, colo1, colo2)
    ab1 = (ab10, ab11, ab12)
    ab2 = (ab20, ab21, ab22)
    gbuf = (gbuf0, gbuf1, gbuf2)
    sem_a1 = (sem_a10, sem_a11, sem_a12)
    sem_a2 = (sem_a20, sem_a21, sem_a22)
    sem_x = (sem_x0, sem_x1, sem_x2)
    sem_s = (sem_s0, sem_s1, sem_s2)

    pltpu.sync_copy(
        xi_hbm.at[pl.ds(c * N + s * RPT, RPT)],
        acc.at[pl.ds(s * RPT, RPT)])

    @pl.when(s == NS - 1)
    def _():
        pltpu.sync_copy(
            xi_hbm.at[pl.ds(c * N + NS * RPT, TAIL)],
            acc.at[pl.ds(NS * RPT, TAIL)])

    plsc.subcore_barrier()

    def prep(u, sl):
        bt = u // BATCH
        jt = lax.rem(u, BATCH)
        sb = lax.rem(bt, 2)
        rb = sb * BATCH + jt

        @pl.when(u >= NSLOT)
        def _():
            pltpu.make_async_copy(
                gbuf[sl], acc.at[row_v.at[rb]], sem_s[sl]).wait()

        @pl.when(jt == 0)
        def _():
            pltpu.make_async_copy(ei_hbm.at[0, s, bt],
                                  row_v.at[pl.ds(sb * BATCH, BATCH)],
                                  sem_ir).wait()
            pltpu.make_async_copy(ei_hbm.at[1, s, bt],
                                  col_v.at[pl.ds(sb * BATCH, BATCH)],
                                  sem_ic).wait()

        @pl.when(jnp.logical_and(jt == 2, bt + 1 < NBATCH))
        def _():
            nb = bt + 1
            nsb = lax.rem(nb, 2)
            pltpu.async_copy(ei_hbm.at[0, s, nb],
                             row_v.at[pl.ds(nsb * BATCH, BATCH)], sem_ir)
            pltpu.async_copy(ei_hbm.at[1, s, nb],
                             col_v.at[pl.ds(nsb * BATCH, BATCH)], sem_ic)

        pltpu.async_copy(a1_hbm.at[row_v.at[rb]], ab1[sl], sem_a1[sl])
        pltpu.async_copy(a2_hbm.at[col_v.at[rb]], ab2[sl], sem_a2[sl])

        for g in range(K // 16):
            colo[sl][pl.ds(g * 16, 16)] = (
                col_v[rb, pl.ds(g * 16, 16)] + coff)

        pltpu.async_copy(xj_hbm.at[colo[sl]], gbuf[sl], sem_x[sl])

    def work(t, sl):
        bt = t // BATCH
        jt = lax.rem(t, BATCH)
        sb = lax.rem(bt, 2)
        rb = sb * BATCH + jt

        pltpu.make_async_copy(a1_hbm.at[row_v.at[rb]], ab1[sl],
                              sem_a1[sl]).wait()
        pltpu.make_async_copy(a2_hbm.at[col_v.at[rb]], ab2[sl],
                              sem_a2[sl]).wait()
        pltpu.make_async_copy(xj_hbm.at[colo[sl]], gbuf[sl],
                              sem_x[sl]).wait()

        def g_body(g, _):
            a1g = ab1[sl][pl.ds(g * 16, 16)]
            a2g = ab2[sl][pl.ds(g * 16, 16)]
            att = 1.0 / (1.0 + jnp.exp(-(a1g + a2g)))
            for l in range(16):
                av = jnp.broadcast_to(att[l], (16,))
                k = g * 16 + l
                for d in range(H // 16):
                    gbuf[sl][k, pl.ds(d * 16, 16)] = (
                        gbuf[sl][k, pl.ds(d * 16, 16)] * av)
            return 0

        lax.fori_loop(0, K // 16, g_body, 0)

        pltpu.async_copy(gbuf[sl], acc.at[row_v.at[rb]], sem_s[sl],
                         add=True)

    pltpu.async_copy(ei_hbm.at[0, s, 0], row_v.at[pl.ds(0, BATCH)], sem_ir)
    pltpu.async_copy(ei_hbm.at[1, s, 0], col_v.at[pl.ds(0, BATCH)], sem_ic)
    prep(jnp.int32(0), 0)
    prep(jnp.int32(1), 1)

    def loop_body(i, _):
        for b3 in range(NSLOT):
            t = NSLOT * i + b3
            pl.when(t < NCHUNK)(lambda: work(t, b3))
            pl.when(t + 2 < NCHUNK)(
                lambda: prep(t + 2, (b3 + 2) % NSLOT))
        return 0

    lax.fori_loop(0, (NCHUNK + NSLOT - 1) // NSLOT, loop_body, 0)

    for x in range(NCHUNK - NSLOT, NCHUNK):
        sl = x % NSLOT
        pltpu.make_async_copy(gbuf[sl], acc.at[row_v.at[0]],
                              sem_s[sl]).wait()

    plsc.subcore_barrier()

    pltpu.sync_copy(
        acc.at[pl.ds(s * RPT, RPT)],
        out_hbm.at[pl.ds(s * RPT, RPT), pl.ds(c * H, H)])

    @pl.when(s == NS - 1)
    def _():
        pltpu.sync_copy(
            acc.at[pl.ds(NS * RPT, TAIL)],
            out_hbm.at[pl.ds(NS * RPT, TAIL), pl.ds(c * H, H)])


@functools.cache
def _sc_segment():
    return pl.kernel(
        _sc_body,
        out_type=jax.ShapeDtypeStruct((N, D), jnp.float32),
        mesh=plsc.VectorSubcoreMesh(core_axis_name="c", subcore_axis_name="s"),
        scratch_types=(
            [
                pltpu.VMEM((2 * BATCH, K), jnp.int32),
                pltpu.VMEM((2 * BATCH, K), jnp.int32),
            ]
            + [pltpu.VMEM((K,), jnp.int32)] * 3
            + [pltpu.VMEM((K,), jnp.float32)] * 6
            + [pltpu.VMEM((K, H), jnp.float32)] * 3
            + [pltpu.VMEM_SHARED((N, H), jnp.float32)]
            + [pltpu.SemaphoreType.DMA] * 14
        ),
        compiler_params=pltpu.CompilerParams(needs_layout_passes=False),
    )


def kernel(x0, x1, edge_index, W1, b1, W2, b2, a1_w, a1_b, a2_w, a2_b):
    ei = edge_index.astype(jnp.int32).reshape(2, NS, NBATCH, BATCH, K)
    xi2, xj2, a1n, a2n = _tc_dense(
        x0, W1.T, W2.T,
        b1.reshape(1, D), b2.reshape(1, D),
        a1_w.T, a2_w.T,
        a1_b.reshape(1, 1), a2_b.reshape(1, 1))
    return _sc_segment()(xj2, xi2, ei, a1n.reshape(N), a2n.reshape(N))


# speedup vs baseline: 1.0948x; 1.0337x over previous
import functools

import jax
import jax.numpy as jnp
from jax import lax
from jax.experimental import pallas as pl
from jax.experimental.pallas import tpu as pltpu
from jax.experimental.pallas import tpu_sc as plsc

N = 10000
E = 160000
D = 256
H = 128
NC = 2
NS = 16
EPT = E // NS
K = 80
NCHUNK = EPT // K
BATCH = 5
NBATCH = NCHUNK // BATCH
NSLOT = 3
RPT = 624
TAIL = N - NS * RPT

BM = 2000
NB = N // BM


_DN_T = (((1,), (1,)), ((), ()))


def _tc_body(x_ref, w1_ref, w2_ref, b1_ref, b2_ref, a1w_ref, a2w_ref,
             a1b_ref, a2b_ref, xi_ref, xj_ref, a1_ref, a2_ref):
    h = pl.program_id(1)
    x = x_ref[...]
    xi = jnp.maximum(
        lax.dot_general(x, w1_ref[...], _DN_T,
                        preferred_element_type=jnp.float32)
        + b1_ref[...], 0.0)
    xj = jnp.maximum(
        lax.dot_general(x, w2_ref[...], _DN_T,
                        preferred_element_type=jnp.float32)
        + b2_ref[...], 0.0)
    xi_ref[...] = xi
    xj_ref[...] = xj
    pa1 = lax.dot_general(xi, a1w_ref[...], _DN_T,
                          preferred_element_type=jnp.float32)
    pa2 = lax.dot_general(xj, a2w_ref[...], _DN_T,
                          preferred_element_type=jnp.float32)

    @pl.when(h == 0)
    def _():
        a1_ref[...] = pa1 + a1b_ref[...]
        a2_ref[...] = pa2 + a2b_ref[...]

    @pl.when(h != 0)
    def _():
        a1_ref[...] += pa1
        a2_ref[...] += pa2


def _tc_dense(x0, w1t, w2t, b1r, b2r, a1wt, a2wt, a1br, a2br):
    return pl.pallas_call(
        _tc_body,
        grid=(NB, 2),
        in_specs=[
            pl.BlockSpec((BM, D), lambda i, h: (i, 0)),
            pl.BlockSpec((H, D), lambda i, h: (h, 0)),
            pl.BlockSpec((H, D), lambda i, h: (h, 0)),
            pl.BlockSpec((1, H), lambda i, h: (0, h)),
            pl.BlockSpec((1, H), lambda i, h: (0, h)),
            pl.BlockSpec((1, H), lambda i, h: (0, h)),
            pl.BlockSpec((1, H), lambda i, h: (0, h)),
            pl.BlockSpec((1, 1), lambda i, h: (0, 0)),
            pl.BlockSpec((1, 1), lambda i, h: (0, 0)),
        ],
        out_specs=[
            pl.BlockSpec((BM, H), lambda i, h: (h * NB + i, 0)),
            pl.BlockSpec((BM, H), lambda i, h: (h * NB + i, 0)),
            pl.BlockSpec((BM, 1), lambda i, h: (i, 0)),
            pl.BlockSpec((BM, 1), lambda i, h: (i, 0)),
        ],
        out_shape=[
            jax.ShapeDtypeStruct((2 * N, H), jnp.float32),
            jax.ShapeDtypeStruct((2 * N, H), jnp.float32),
            jax.ShapeDtypeStruct((N, 1), jnp.float32),
            jax.ShapeDtypeStruct((N, 1), jnp.float32),
        ],
    )(x0, w1t, w2t, b1r, b2r, a1wt, a2wt, a1br, a2br)


def _sc_body(xj_hbm, xi_hbm, ei_hbm, a1_hbm, a2_hbm, out_hbm,
             row_v, col_v,
             colo0, colo1, colo2, ab10, ab11, ab12, ab20, ab21, ab22,
             gbuf0, gbuf1, gbuf2, acc,
             sem_ir, sem_ic,
             sem_a10, sem_a11, sem_a12, sem_a20, sem_a21, sem_a22,
             sem_x0, sem_x1, sem_x2, sem_s0, sem_s1, sem_s2):
    c = lax.axis_index("c")
    s = lax.axis_index("s")
    coff = c * N

    colo = (colo0, colo1, colo2)
    ab1 = (ab10, ab11, ab12)
    ab2 = (ab20, ab21, ab22)
    gbuf = (gbuf0, gbuf1, gbuf2)
    sem_a1 = (sem_a10, sem_a11, sem_a12)
    sem_a2 = (sem_a20, sem_a21, sem_a22)
    sem_x = (sem_x0, sem_x1, sem_x2)
    sem_s = (sem_s0, sem_s1, sem_s2)

    pltpu.sync_copy(
        xi_hbm.at[pl.ds(c * N + s * RPT, RPT)],
        acc.at[pl.ds(s * RPT, RPT)])

    @pl.when(s == NS - 1)
    def _():
        pltpu.sync_copy(
            xi_hbm.at[pl.ds(c * N + NS * RPT, TAIL)],
            acc.at[pl.ds(NS * RPT, TAIL)])

    plsc.subcore_barrier()

    def prep(u, sl):
        bt = u // BATCH
        jt = lax.rem(u, BATCH)
        sb = lax.rem(bt, 2)
        rb = sb * BATCH + jt

        @pl.when(u >= NSLOT)
        def _():
            pltpu.make_async_copy(
                gbuf[sl], acc.at[row_v.at[rb]], sem_s[sl]).wait()

        @pl.when(jt == 0)
        def _():
            pltpu.make_async_copy(ei_hbm.at[0, s, bt],
                                  row_v.at[pl.ds(sb * BATCH, BATCH)],
                                  sem_ir).wait()
            pltpu.make_async_copy(ei_hbm.at[1, s, bt],
                                  col_v.at[pl.ds(sb * BATCH, BATCH)],
                                  sem_ic).wait()

        @pl.when(jnp.logical_and(jt == 2, bt + 1 < NBATCH))
        def _():
            nb = bt + 1
            nsb = lax.rem(nb, 2)
            pltpu.async_copy(ei_hbm.at[0, s, nb],
                             row_v.at[pl.ds(nsb * BATCH, BATCH)], sem_ir)
            pltpu.async_copy(ei_hbm.at[1, s, nb],
                             col_v.at[pl.ds(nsb * BATCH, BATCH)], sem_ic)

        pltpu.async_copy(a1_hbm.at[row_v.at[rb]], ab1[sl], sem_a1[sl])
        pltpu.async_copy(a2_hbm.at[col_v.at[rb]], ab2[sl], sem_a2[sl])

        for g in range(K // 16):
            colo[sl][pl.ds(g * 16, 16)] = (
                col_v[rb, pl.ds(g * 16, 16)] + coff)

        pltpu.async_copy(xj_hbm.at[colo[sl]], gbuf[sl], sem_x[sl])

    def work(t, sl):
        bt = t // BATCH
        jt = lax.rem(t, BATCH)
        sb = lax.rem(bt, 2)
        rb = sb * BATCH + jt

        pltpu.make_async_copy(a1_hbm.at[row_v.at[rb]], ab1[sl],
                              sem_a1[sl]).wait()
        pltpu.make_async_copy(a2_hbm.at[col_v.at[rb]], ab2[sl],
                              sem_a2[sl]).wait()
        pltpu.make_async_copy(xj_hbm.at[colo[sl]], gbuf[sl],
                              sem_x[sl]).wait()

        def g_body(g, _):
            a1g = ab1[sl][pl.ds(g * 16, 16)]
            a2g = ab2[sl][pl.ds(g * 16, 16)]
            att = 1.0 / (1.0 + jnp.exp(-(a1g + a2g)))
            for l in range(16):
                av = jnp.broadcast_to(att[l], (16,))
                k = g * 16 + l
                for d in range(H // 16):
                    gbuf[sl][k, pl.ds(d * 16, 16)] = (
                        gbuf[sl][k, pl.ds(d * 16, 16)] * av)
            return 0

        lax.fori_loop(0, K // 16, g_body, 0)

        pltpu.async_copy(gbuf[sl], acc.at[row_v.at[rb]], sem_s[sl],
                         add=True)

    pltpu.async_copy(ei_hbm.at[0, s, 0], row_v.at[pl.ds(0, BATCH)], sem_ir)
    pltpu.async_copy(ei_hbm.at[1, s, 0], col_v.at[pl.ds(0, BATCH)], sem_ic)
    prep(jnp.int32(0), 0)
    prep(jnp.int32(1), 1)

    def loop_body(i, _):
        for b3 in range(NSLOT):
            t = NSLOT * i + b3
            pl.when(t < NCHUNK)(lambda: work(t, b3))
            pl.when(t + 2 < NCHUNK)(
                lambda: prep(t + 2, (b3 + 2) % NSLOT))
        return 0

    lax.fori_loop(0, (NCHUNK + NSLOT - 1) // NSLOT, loop_body, 0)

    for x in range(NCHUNK - NSLOT, NCHUNK):
        sl = x % NSLOT
        pltpu.make_async_copy(gbuf[sl], acc.at[row_v.at[0]],
                              sem_s[sl]).wait()

    plsc.subcore_barrier()

    pltpu.sync_copy(
        acc.at[pl.ds(s * RPT, RPT)],
        out_hbm.at[pl.ds(s * RPT, RPT), pl.ds(c * H, H)])

    @pl.when(s == NS - 1)
    def _():
        pltpu.sync_copy(
            acc.at[pl.ds(NS * RPT, TAIL)],
            out_hbm.at[pl.ds(NS * RPT, TAIL), pl.ds(c * H, H)])


@functools.cache
def _sc_segment():
    return pl.kernel(
        _sc_body,
        out_type=jax.ShapeDtypeStruct((N, D), jnp.float32),
        mesh=plsc.VectorSubcoreMesh(core_axis_name="c", subcore_axis_name="s"),
        scratch_types=(
            [
                pltpu.VMEM((2 * BATCH, K), jnp.int32),
                pltpu.VMEM((2 * BATCH, K), jnp.int32),
            ]
            + [pltpu.VMEM((K,), jnp.int32)] * 3
            + [pltpu.VMEM((K,), jnp.float32)] * 6
            + [pltpu.VMEM((K, H), jnp.float32)] * 3
            + [pltpu.VMEM_SHARED((N, H), jnp.float32)]
            + [pltpu.SemaphoreType.DMA] * 14
        ),
        compiler_params=pltpu.CompilerParams(needs_layout_passes=False),
    )


def kernel(x0, x1, edge_index, W1, b1, W2, b2, a1_w, a1_b, a2_w, a2_b):
    ei = edge_index.astype(jnp.int32).reshape(2, NS, NBATCH, BATCH, K)
    xi2, xj2, a1n, a2n = _tc_dense(
        x0, W1, W2,
        b1.reshape(1, D), b2.reshape(1, D),
        a1_w, a2_w,
        a1_b.reshape(1, 1), a2_b.reshape(1, 1))
    return _sc_segment()(xj2, xi2, ei, a1n.reshape(N), a2n.reshape(N))
